# Initial kernel scaffold; baseline (speedup 1.0000x reference)
import jax
import jax.numpy as jnp
from jax.experimental import pallas as pl


def kernel(x, W, expert_bias, Wh, hash_tables, return_raw_logits):
    raise NotImplementedError("write your pallas kernel here")


# speedup vs baseline: 2.6507x; 2.6507x over previous
import jax
import jax.numpy as jnp
import numpy as np
from jax import lax
from jax.experimental import pallas as pl
from jax.experimental.pallas import tpu as pltpu
from jax.experimental.pallas import tpu_sc as plsc

D_MODEL = 768
NUM_EXPERTS = 64
TOP_K = 8
NUM_BUCKETS = 64
BUCKET_SIZE = 8
N_TOKENS = 32768
CLAMP_MIN = -10000.0
CLAMP_MAX = 10000.0

BT = 1024

NC = 2
NS = 16
NW = NC * NS
TOK_PER_W = N_TOKENS // NW


_THR = np.log([k / (64.0 - k) for k in range(1, 64)])
_TLO = np.concatenate([[-1e30], _THR]).astype(np.float32).reshape(1, 64)
_THI = np.concatenate([_THR, [1e30]]).astype(np.float32).reshape(1, 64)


def _rne_bf16(x):
    u = lax.bitcast_convert_type(x, jnp.uint32)
    u = (u + jnp.uint32(0x7FFF) + ((u >> jnp.uint32(16)) & jnp.uint32(1)))
    u = u & jnp.uint32(0xFFFF0000)
    return lax.bitcast_convert_type(u, jnp.float32)


def _tc_body(x_ref, w_ref, b_ref, wh0_ref, wh1_ref, ht0_ref, ht1_ref,
             tlo_ref, thi_ref, logits_ref, cand_ref):
    xb = x_ref[...]
    acc = lax.dot_general(
        xb, w_ref[...], (((1,), (1,)), ((), ())),
        preferred_element_type=jnp.float32,
    )
    logits = jnp.clip(acc + b_ref[...], CLAMP_MIN, CLAMP_MAX)
    logits_ref[...] = logits

    xq = _rne_bf16(xb)
    tlo = tlo_ref[...]
    thi = thi_ref[...]
    cols = []
    for wh_ref, ht_ref in ((wh0_ref, ht0_ref), (wh1_ref, ht1_ref)):
        hl = lax.dot_general(
            xq, _rne_bf16(wh_ref[...]), (((1,), (1,)), ((), ())),
            preferred_element_type=jnp.float32,
        )
        onehot = ((hl >= tlo) & (hl < thi)).astype(jnp.float32)
        ch = lax.dot_general(
            onehot, ht_ref[...], (((1,), (0,)), ((), ())),
            preferred_element_type=jnp.float32,
        )
        cols.append(ch)
    cand_ref[...] = jnp.concatenate(cols, axis=1).astype(jnp.int32)


@jax.jit
def _tc_stage(x, W, bias2d, wh0, wh1, ht0f, ht1f):
    grid = (N_TOKENS // BT,)
    return pl.pallas_call(
        _tc_body,
        grid=grid,
        in_specs=[
            pl.BlockSpec((BT, D_MODEL), lambda i: (i, 0)),
            pl.BlockSpec((NUM_EXPERTS, D_MODEL), lambda i: (0, 0)),
            pl.BlockSpec((1, NUM_EXPERTS), lambda i: (0, 0)),
            pl.BlockSpec((1, D_MODEL), lambda i: (0, 0)),
            pl.BlockSpec((1, D_MODEL), lambda i: (0, 0)),
            pl.BlockSpec((NUM_BUCKETS, BUCKET_SIZE), lambda i: (0, 0)),
            pl.BlockSpec((NUM_BUCKETS, BUCKET_SIZE), lambda i: (0, 0)),
            pl.BlockSpec((1, NUM_BUCKETS), lambda i: (0, 0)),
            pl.BlockSpec((1, NUM_BUCKETS), lambda i: (0, 0)),
        ],
        out_specs=[
            pl.BlockSpec((BT, NUM_EXPERTS), lambda i: (i, 0)),
            pl.BlockSpec((BT, 16), lambda i: (i, 0)),
        ],
        out_shape=[
            jax.ShapeDtypeStruct((N_TOKENS, NUM_EXPERTS), jnp.float32),
            jax.ShapeDtypeStruct((N_TOKENS, 16), jnp.int32),
        ],
        compiler_params=pltpu.CompilerParams(
            dimension_semantics=("arbitrary",),
        ),
    )(x, W, bias2d, wh0, wh1, ht0f, ht1f,
      jnp.asarray(_TLO), jnp.asarray(_THI))


def _sc_body(logits_hbm, cand_hbm, outp_hbm, outi_hbm,
             logits_v, cand_v, outp_v, outi_v):
    wid = lax.axis_index("s") * NC + lax.axis_index("c")
    base = wid * TOK_PER_W
    pltpu.sync_copy(logits_hbm.at[pl.ds(base * NUM_EXPERTS,
                                        TOK_PER_W * NUM_EXPERTS)], logits_v)
    pltpu.sync_copy(cand_hbm.at[pl.ds(base * 16, TOK_PER_W * 16)], cand_v)

    lane = lax.iota(jnp.int32, 16)
    topmask = lane < TOP_K

    def body(t, carry):
        cand16 = cand_v[pl.ds(t * 16, 16)]
        sc = plsc.load_gather(logits_v, [cand16 + t * NUM_EXPERTS])
        ks, vs = plsc.sort_key_val(sc, cand16, descending=True)
        m = jnp.max(sc)
        e = jnp.where(topmask, jnp.exp(ks - m), 0.0)
        s = jnp.sum(e)
        outp_v[pl.ds(t * 16, 16)] = e / (s + 1e-12)
        outi_v[pl.ds(t * 16, 16)] = vs
        return carry

    lax.fori_loop(0, TOK_PER_W, body, 0)

    pltpu.sync_copy(outp_v, outp_hbm.at[pl.ds(base * 16, TOK_PER_W * 16)])
    pltpu.sync_copy(outi_v, outi_hbm.at[pl.ds(base * 16, TOK_PER_W * 16)])


_sc_stage = jax.jit(pl.kernel(
    _sc_body,
    out_type=(
        jax.ShapeDtypeStruct((N_TOKENS * 16,), jnp.float32),
        jax.ShapeDtypeStruct((N_TOKENS * 16,), jnp.int32),
    ),
    mesh=plsc.VectorSubcoreMesh(core_axis_name="c", subcore_axis_name="s"),
    compiler_params=pltpu.CompilerParams(needs_layout_passes=False),
    scratch_types=[
        pltpu.VMEM((TOK_PER_W * NUM_EXPERTS,), jnp.float32),
        pltpu.VMEM((TOK_PER_W * 16,), jnp.int32),
        pltpu.VMEM((TOK_PER_W * 16,), jnp.float32),
        pltpu.VMEM((TOK_PER_W * 16,), jnp.int32),
    ],
))


def kernel(x, W, expert_bias, Wh, hash_tables, return_raw_logits=1):
    bias2d = expert_bias.reshape(1, NUM_EXPERTS)
    wh0 = Wh[0:1, :]
    wh1 = Wh[1:2, :]
    ht0f = hash_tables[0].astype(jnp.float32)
    ht1f = hash_tables[1].astype(jnp.float32)
    logits, cand = _tc_stage(x, W, bias2d, wh0, wh1, ht0f, ht1f)
    p16, i16 = _sc_stage(logits.reshape(-1), cand.reshape(-1))
    top_p = p16.reshape(N_TOKENS, 16)[:, :TOP_K]
    top_i = i16.reshape(N_TOKENS, 16)[:, :TOP_K]
    return (logits, logits, logits, top_p, top_i)


# speedup vs baseline: 3.2115x; 1.2116x over previous
import jax
import jax.numpy as jnp
import numpy as np
from jax import lax
from jax.experimental import pallas as pl
from jax.experimental.pallas import tpu as pltpu
from jax.experimental.pallas import tpu_sc as plsc

D_MODEL = 768
NUM_EXPERTS = 64
TOP_K = 8
NUM_BUCKETS = 64
BUCKET_SIZE = 8
N_TOKENS = 32768
CLAMP_MIN = -10000.0
CLAMP_MAX = 10000.0

BT = 1024

NC = 2
NS = 16
NW = NC * NS
TOK_PER_W = N_TOKENS // NW


_THR = np.log([k / (64.0 - k) for k in range(1, 64)])
_TLO = np.concatenate([[-1e30], _THR]).astype(np.float32).reshape(1, 64)
_THI = np.concatenate([_THR, [1e30]]).astype(np.float32).reshape(1, 64)


def _rne_bf16(x):
    u = lax.bitcast_convert_type(x, jnp.uint32)
    u = (u + jnp.uint32(0x7FFF) + ((u >> jnp.uint32(16)) & jnp.uint32(1)))
    u = u & jnp.uint32(0xFFFF0000)
    return lax.bitcast_convert_type(u, jnp.float32)


def _tc_body(x_ref, w_ref, b_ref, wh0_ref, wh1_ref, ht0_ref, ht1_ref,
             tlo_ref, thi_ref, logits_ref, cand_ref):
    xb = x_ref[...]
    acc = lax.dot_general(
        xb, w_ref[...], (((1,), (1,)), ((), ())),
        preferred_element_type=jnp.float32,
    )
    logits = jnp.clip(acc + b_ref[...], CLAMP_MIN, CLAMP_MAX)
    logits_ref[...] = logits

    xq = _rne_bf16(xb)
    tlo = tlo_ref[...]
    thi = thi_ref[...]
    cols = []
    for wh_ref, ht_ref in ((wh0_ref, ht0_ref), (wh1_ref, ht1_ref)):
        hl = lax.dot_general(
            xq, _rne_bf16(wh_ref[...]), (((1,), (1,)), ((), ())),
            preferred_element_type=jnp.float32,
        )
        onehot = ((hl >= tlo) & (hl < thi)).astype(jnp.float32)
        ch = lax.dot_general(
            onehot, ht_ref[...], (((1,), (0,)), ((), ())),
            preferred_element_type=jnp.float32,
        )
        cols.append(ch)
    cand_ref[...] = jnp.concatenate(cols, axis=1).astype(jnp.int32)


@jax.jit
def _tc_stage(x, W, bias2d, wh0, wh1, ht0f, ht1f):
    grid = (N_TOKENS // BT,)
    return pl.pallas_call(
        _tc_body,
        grid=grid,
        in_specs=[
            pl.BlockSpec((BT, D_MODEL), lambda i: (i, 0)),
            pl.BlockSpec((NUM_EXPERTS, D_MODEL), lambda i: (0, 0)),
            pl.BlockSpec((1, NUM_EXPERTS), lambda i: (0, 0)),
            pl.BlockSpec((1, D_MODEL), lambda i: (0, 0)),
            pl.BlockSpec((1, D_MODEL), lambda i: (0, 0)),
            pl.BlockSpec((NUM_BUCKETS, BUCKET_SIZE), lambda i: (0, 0)),
            pl.BlockSpec((NUM_BUCKETS, BUCKET_SIZE), lambda i: (0, 0)),
            pl.BlockSpec((1, NUM_BUCKETS), lambda i: (0, 0)),
            pl.BlockSpec((1, NUM_BUCKETS), lambda i: (0, 0)),
        ],
        out_specs=[
            pl.BlockSpec((BT, NUM_EXPERTS), lambda i: (i, 0)),
            pl.BlockSpec((BT, 16), lambda i: (i, 0)),
        ],
        out_shape=[
            jax.ShapeDtypeStruct((N_TOKENS, NUM_EXPERTS), jnp.float32),
            jax.ShapeDtypeStruct((N_TOKENS, 16), jnp.int32),
        ],
        compiler_params=pltpu.CompilerParams(
            dimension_semantics=("arbitrary",),
        ),
    )(x, W, bias2d, wh0, wh1, ht0f, ht1f,
      jnp.asarray(_TLO), jnp.asarray(_THI))


def _sc_body(logits_hbm, cand_hbm, outp_hbm, outi_hbm,
             logits_v, cand_v, outp_v, outi_v):
    wid = lax.axis_index("s") * NC + lax.axis_index("c")
    base = wid * TOK_PER_W
    pltpu.sync_copy(logits_hbm.at[pl.ds(base * NUM_EXPERTS,
                                        TOK_PER_W * NUM_EXPERTS)], logits_v)
    pltpu.sync_copy(cand_hbm.at[pl.ds(base * 16, TOK_PER_W * 16)], cand_v)

    lane = lax.iota(jnp.int32, 16)
    topmask = lane < TOP_K

    @plsc.parallel_loop(0, TOK_PER_W, step=1, unroll=8)
    def body(t):
        cand16 = cand_v[pl.ds(t * 16, 16)]
        sc = plsc.load_gather(logits_v, [cand16 + t * NUM_EXPERTS])
        ks, vs = plsc.sort_key_val(sc, cand16, descending=True)
        m = jnp.max(ks)
        e = jnp.where(topmask, jnp.exp(ks - m), 0.0)
        s = jnp.sum(e)
        plsc.store_compressed(outp_v.at[pl.ds(t * TOP_K, 16)], e / (s + 1e-12),
                              mask=topmask)
        plsc.store_compressed(outi_v.at[pl.ds(t * TOP_K, 16)], vs, mask=topmask)

    pltpu.sync_copy(outp_v.at[pl.ds(0, TOK_PER_W * TOP_K)],
                    outp_hbm.at[pl.ds(base * TOP_K, TOK_PER_W * TOP_K)])
    pltpu.sync_copy(outi_v.at[pl.ds(0, TOK_PER_W * TOP_K)],
                    outi_hbm.at[pl.ds(base * TOP_K, TOK_PER_W * TOP_K)])


_sc_stage = jax.jit(pl.kernel(
    _sc_body,
    out_type=(
        jax.ShapeDtypeStruct((N_TOKENS * TOP_K,), jnp.float32),
        jax.ShapeDtypeStruct((N_TOKENS * TOP_K,), jnp.int32),
    ),
    mesh=plsc.VectorSubcoreMesh(core_axis_name="c", subcore_axis_name="s"),
    compiler_params=pltpu.CompilerParams(needs_layout_passes=False),
    scratch_types=[
        pltpu.VMEM((TOK_PER_W * NUM_EXPERTS,), jnp.float32),
        pltpu.VMEM((TOK_PER_W * 16,), jnp.int32),
        pltpu.VMEM((TOK_PER_W * TOP_K + 8,), jnp.float32),
        pltpu.VMEM((TOK_PER_W * TOP_K + 8,), jnp.int32),
    ],
))


def kernel(x, W, expert_bias, Wh, hash_tables, return_raw_logits=1):
    bias2d = expert_bias.reshape(1, NUM_EXPERTS)
    wh0 = Wh[0:1, :]
    wh1 = Wh[1:2, :]
    ht0f = hash_tables[0].astype(jnp.float32)
    ht1f = hash_tables[1].astype(jnp.float32)
    logits, cand = _tc_stage(x, W, bias2d, wh0, wh1, ht0f, ht1f)
    p8, i8 = _sc_stage(logits.reshape(-1), cand.reshape(-1))
    top_p = p8.reshape(N_TOKENS, TOP_K)
    top_i = i8.reshape(N_TOKENS, TOP_K)
    return (logits, logits, logits, top_p, top_i)


# speedup vs baseline: 3.3513x; 1.0435x over previous
import jax
import jax.numpy as jnp
import numpy as np
from jax import lax
from jax.experimental import pallas as pl
from jax.experimental.pallas import tpu as pltpu
from jax.experimental.pallas import tpu_sc as plsc

D_MODEL = 768
NUM_EXPERTS = 64
TOP_K = 8
NUM_BUCKETS = 64
BUCKET_SIZE = 8
N_TOKENS = 32768
CLAMP_MIN = -10000.0
CLAMP_MAX = 10000.0

BT = 2048

NC = 2
NS = 16
NW = NC * NS
TOK_PER_W = N_TOKENS // NW


_THR = np.log([k / (64.0 - k) for k in range(1, 64)])
_TLO = np.concatenate([[-1e30], _THR]).astype(np.float32).reshape(1, 64)
_THI = np.concatenate([_THR, [1e30]]).astype(np.float32).reshape(1, 64)


def _rne_bf16(x):
    u = lax.bitcast_convert_type(x, jnp.uint32)
    u = (u + jnp.uint32(0x7FFF) + ((u >> jnp.uint32(16)) & jnp.uint32(1)))
    u = u & jnp.uint32(0xFFFF0000)
    return lax.bitcast_convert_type(u, jnp.float32)


def _tc_body(x_ref, w_ref, b_ref, wh0_ref, wh1_ref, ht0_ref, ht1_ref,
             tlo_ref, thi_ref, logits_ref, cand_ref):
    xb = x_ref[...]
    acc = lax.dot_general(
        xb, w_ref[...], (((1,), (1,)), ((), ())),
        preferred_element_type=jnp.float32,
    )
    logits = jnp.clip(acc + b_ref[...], CLAMP_MIN, CLAMP_MAX)
    logits_ref[...] = logits

    xq = _rne_bf16(xb)
    tlo = tlo_ref[...]
    thi = thi_ref[...]
    cols = []
    for wh_ref, ht_ref in ((wh0_ref, ht0_ref), (wh1_ref, ht1_ref)):
        hl = lax.dot_general(
            xq, _rne_bf16(wh_ref[...]), (((1,), (1,)), ((), ())),
            preferred_element_type=jnp.float32,
        )
        onehot = ((hl >= tlo) & (hl < thi)).astype(jnp.float32)
        ch = lax.dot_general(
            onehot, ht_ref[...], (((1,), (0,)), ((), ())),
            preferred_element_type=jnp.float32,
        )
        cols.append(ch)
    cand_ref[...] = jnp.concatenate(cols, axis=1).astype(jnp.int32)


@jax.jit
def _tc_stage(x, W, bias2d, wh0, wh1, ht0f, ht1f):
    grid = (N_TOKENS // BT,)
    return pl.pallas_call(
        _tc_body,
        grid=grid,
        in_specs=[
            pl.BlockSpec((BT, D_MODEL), lambda i: (i, 0)),
            pl.BlockSpec((NUM_EXPERTS, D_MODEL), lambda i: (0, 0)),
            pl.BlockSpec((1, NUM_EXPERTS), lambda i: (0, 0)),
            pl.BlockSpec((1, D_MODEL), lambda i: (0, 0)),
            pl.BlockSpec((1, D_MODEL), lambda i: (0, 0)),
            pl.BlockSpec((NUM_BUCKETS, BUCKET_SIZE), lambda i: (0, 0)),
            pl.BlockSpec((NUM_BUCKETS, BUCKET_SIZE), lambda i: (0, 0)),
            pl.BlockSpec((1, NUM_BUCKETS), lambda i: (0, 0)),
            pl.BlockSpec((1, NUM_BUCKETS), lambda i: (0, 0)),
        ],
        out_specs=[
            pl.BlockSpec((BT, NUM_EXPERTS), lambda i: (i, 0)),
            pl.BlockSpec((BT, 16), lambda i: (i, 0)),
        ],
        out_shape=[
            jax.ShapeDtypeStruct((N_TOKENS, NUM_EXPERTS), jnp.float32),
            jax.ShapeDtypeStruct((N_TOKENS, 16), jnp.int32),
        ],
        compiler_params=pltpu.CompilerParams(
            dimension_semantics=("arbitrary",),
        ),
    )(x, W, bias2d, wh0, wh1, ht0f, ht1f,
      jnp.asarray(_TLO), jnp.asarray(_THI))


def _sc_body(logits_hbm, cand_hbm, outp_hbm, outi_hbm,
             logits_v, cand_v, outp_v, outi_v):
    wid = lax.axis_index("s") * NC + lax.axis_index("c")
    base = wid * TOK_PER_W
    pltpu.sync_copy(logits_hbm.at[pl.ds(base * NUM_EXPERTS,
                                        TOK_PER_W * NUM_EXPERTS)], logits_v)
    pltpu.sync_copy(cand_hbm.at[pl.ds(base * 16, TOK_PER_W * 16)], cand_v)

    lane = lax.iota(jnp.int32, 16)
    topmask = lane < TOP_K

    @plsc.parallel_loop(0, TOK_PER_W, step=1, unroll=8)
    def body(t):
        cand16 = cand_v[pl.ds(t * 16, 16)]
        sc = plsc.load_gather(logits_v, [cand16 + t * NUM_EXPERTS])
        ks, vs = plsc.sort_key_val(sc, cand16, descending=True)
        m = jnp.max(ks)
        e = jnp.where(topmask, jnp.exp(ks - m), 0.0)
        s = jnp.sum(e)
        plsc.store_compressed(outp_v.at[pl.ds(t * TOP_K, 16)], e / (s + 1e-12),
                              mask=topmask)
        plsc.store_compressed(outi_v.at[pl.ds(t * TOP_K, 16)], vs, mask=topmask)

    pltpu.sync_copy(outp_v.at[pl.ds(0, TOK_PER_W * TOP_K)],
                    outp_hbm.at[pl.ds(base * TOP_K, TOK_PER_W * TOP_K)])
    pltpu.sync_copy(outi_v.at[pl.ds(0, TOK_PER_W * TOP_K)],
                    outi_hbm.at[pl.ds(base * TOP_K, TOK_PER_W * TOP_K)])


_sc_stage = jax.jit(pl.kernel(
    _sc_body,
    out_type=(
        jax.ShapeDtypeStruct((N_TOKENS * TOP_K,), jnp.float32),
        jax.ShapeDtypeStruct((N_TOKENS * TOP_K,), jnp.int32),
    ),
    mesh=plsc.VectorSubcoreMesh(core_axis_name="c", subcore_axis_name="s"),
    compiler_params=pltpu.CompilerParams(needs_layout_passes=False),
    scratch_types=[
        pltpu.VMEM((TOK_PER_W * NUM_EXPERTS,), jnp.float32),
        pltpu.VMEM((TOK_PER_W * 16,), jnp.int32),
        pltpu.VMEM((TOK_PER_W * TOP_K + 8,), jnp.float32),
        pltpu.VMEM((TOK_PER_W * TOP_K + 8,), jnp.int32),
    ],
))


def kernel(x, W, expert_bias, Wh, hash_tables, return_raw_logits=1):
    bias2d = expert_bias.reshape(1, NUM_EXPERTS)
    wh0 = Wh[0:1, :]
    wh1 = Wh[1:2, :]
    ht0f = hash_tables[0].astype(jnp.float32)
    ht1f = hash_tables[1].astype(jnp.float32)
    logits, cand = _tc_stage(x, W, bias2d, wh0, wh1, ht0f, ht1f)
    p8, i8 = _sc_stage(logits.reshape(-1), cand.reshape(-1))
    top_p = p8.reshape(N_TOKENS, TOP_K)
    top_i = i8.reshape(N_TOKENS, TOP_K)
    return (logits, logits, logits, top_p, top_i)


# speedup vs baseline: 3.4301x; 1.0235x over previous
import jax
import jax.numpy as jnp
import numpy as np
from jax import lax
from jax.experimental import pallas as pl
from jax.experimental.pallas import tpu as pltpu
from jax.experimental.pallas import tpu_sc as plsc

D_MODEL = 768
NUM_EXPERTS = 64
TOP_K = 8
NUM_BUCKETS = 64
BUCKET_SIZE = 8
N_TOKENS = 32768
CLAMP_MIN = -10000.0
CLAMP_MAX = 10000.0

BT = 4096

NC = 2
NS = 16
NW = NC * NS
TOK_PER_W = N_TOKENS // NW


_THR = np.log([k / (64.0 - k) for k in range(1, 64)])
_TLO = np.concatenate([[-1e30], _THR]).astype(np.float32).reshape(1, 64)
_THI = np.concatenate([_THR, [1e30]]).astype(np.float32).reshape(1, 64)


def _rne_bf16(x):
    u = lax.bitcast_convert_type(x, jnp.uint32)
    u = (u + jnp.uint32(0x7FFF) + ((u >> jnp.uint32(16)) & jnp.uint32(1)))
    u = u & jnp.uint32(0xFFFF0000)
    return lax.bitcast_convert_type(u, jnp.float32)


def _tc_body(x_ref, w_ref, b_ref, wh0_ref, wh1_ref, ht0_ref, ht1_ref,
             tlo_ref, thi_ref, logits_ref, cand_ref):
    xb = x_ref[...]
    acc = lax.dot_general(
        xb, w_ref[...], (((1,), (1,)), ((), ())),
        preferred_element_type=jnp.float32,
    )
    logits = jnp.clip(acc + b_ref[...], CLAMP_MIN, CLAMP_MAX)
    logits_ref[...] = logits

    xq = _rne_bf16(xb)
    tlo = tlo_ref[...]
    thi = thi_ref[...]
    cols = []
    for wh_ref, ht_ref in ((wh0_ref, ht0_ref), (wh1_ref, ht1_ref)):
        hl = lax.dot_general(
            xq, _rne_bf16(wh_ref[...]), (((1,), (1,)), ((), ())),
            preferred_element_type=jnp.float32,
        )
        onehot = ((hl >= tlo) & (hl < thi)).astype(jnp.float32)
        ch = lax.dot_general(
            onehot, ht_ref[...], (((1,), (0,)), ((), ())),
            preferred_element_type=jnp.float32,
        )
        cols.append(ch)
    cand_ref[...] = jnp.concatenate(cols, axis=1).astype(jnp.int32)


@jax.jit
def _tc_stage(x, W, bias2d, wh0, wh1, ht0f, ht1f):
    grid = (N_TOKENS // BT,)
    return pl.pallas_call(
        _tc_body,
        grid=grid,
        in_specs=[
            pl.BlockSpec((BT, D_MODEL), lambda i: (i, 0)),
            pl.BlockSpec((NUM_EXPERTS, D_MODEL), lambda i: (0, 0)),
            pl.BlockSpec((1, NUM_EXPERTS), lambda i: (0, 0)),
            pl.BlockSpec((1, D_MODEL), lambda i: (0, 0)),
            pl.BlockSpec((1, D_MODEL), lambda i: (0, 0)),
            pl.BlockSpec((NUM_BUCKETS, BUCKET_SIZE), lambda i: (0, 0)),
            pl.BlockSpec((NUM_BUCKETS, BUCKET_SIZE), lambda i: (0, 0)),
            pl.BlockSpec((1, NUM_BUCKETS), lambda i: (0, 0)),
            pl.BlockSpec((1, NUM_BUCKETS), lambda i: (0, 0)),
        ],
        out_specs=[
            pl.BlockSpec((BT, NUM_EXPERTS), lambda i: (i, 0)),
            pl.BlockSpec((BT, 16), lambda i: (i, 0)),
        ],
        out_shape=[
            jax.ShapeDtypeStruct((N_TOKENS, NUM_EXPERTS), jnp.float32),
            jax.ShapeDtypeStruct((N_TOKENS, 16), jnp.int32),
        ],
        compiler_params=pltpu.CompilerParams(
            dimension_semantics=("arbitrary",),
        ),
    )(x, W, bias2d, wh0, wh1, ht0f, ht1f,
      jnp.asarray(_TLO), jnp.asarray(_THI))


def _sc_body(logits_hbm, cand_hbm, outp_hbm, outi_hbm,
             logits_v, cand_v, outp_v, outi_v):
    wid = lax.axis_index("s") * NC + lax.axis_index("c")
    base = wid * TOK_PER_W
    pltpu.sync_copy(logits_hbm.at[pl.ds(base * NUM_EXPERTS,
                                        TOK_PER_W * NUM_EXPERTS)], logits_v)
    pltpu.sync_copy(cand_hbm.at[pl.ds(base * 16, TOK_PER_W * 16)], cand_v)

    lane = lax.iota(jnp.int32, 16)
    topmask = lane < TOP_K

    @plsc.parallel_loop(0, TOK_PER_W, step=1, unroll=8)
    def body(t):
        cand16 = cand_v[pl.ds(t * 16, 16)]
        sc = plsc.load_gather(logits_v, [cand16 + t * NUM_EXPERTS])
        ks, vs = plsc.sort_key_val(sc, cand16, descending=True)
        m = jnp.max(ks)
        e = jnp.where(topmask, jnp.exp(ks - m), 0.0)
        s = jnp.sum(e)
        plsc.store_compressed(outp_v.at[pl.ds(t * TOP_K, 16)], e / (s + 1e-12),
                              mask=topmask)
        plsc.store_compressed(outi_v.at[pl.ds(t * TOP_K, 16)], vs, mask=topmask)

    pltpu.sync_copy(outp_v.at[pl.ds(0, TOK_PER_W * TOP_K)],
                    outp_hbm.at[pl.ds(base * TOP_K, TOK_PER_W * TOP_K)])
    pltpu.sync_copy(outi_v.at[pl.ds(0, TOK_PER_W * TOP_K)],
                    outi_hbm.at[pl.ds(base * TOP_K, TOK_PER_W * TOP_K)])


_sc_stage = jax.jit(pl.kernel(
    _sc_body,
    out_type=(
        jax.ShapeDtypeStruct((N_TOKENS * TOP_K,), jnp.float32),
        jax.ShapeDtypeStruct((N_TOKENS * TOP_K,), jnp.int32),
    ),
    mesh=plsc.VectorSubcoreMesh(core_axis_name="c", subcore_axis_name="s"),
    compiler_params=pltpu.CompilerParams(needs_layout_passes=False),
    scratch_types=[
        pltpu.VMEM((TOK_PER_W * NUM_EXPERTS,), jnp.float32),
        pltpu.VMEM((TOK_PER_W * 16,), jnp.int32),
        pltpu.VMEM((TOK_PER_W * TOP_K + 8,), jnp.float32),
        pltpu.VMEM((TOK_PER_W * TOP_K + 8,), jnp.int32),
    ],
))


def kernel(x, W, expert_bias, Wh, hash_tables, return_raw_logits=1):
    bias2d = expert_bias.reshape(1, NUM_EXPERTS)
    wh0 = Wh[0:1, :]
    wh1 = Wh[1:2, :]
    ht0f = hash_tables[0].astype(jnp.float32)
    ht1f = hash_tables[1].astype(jnp.float32)
    logits, cand = _tc_stage(x, W, bias2d, wh0, wh1, ht0f, ht1f)
    p8, i8 = _sc_stage(logits.reshape(-1), cand.reshape(-1))
    top_p = p8.reshape(N_TOKENS, TOP_K)
    top_i = i8.reshape(N_TOKENS, TOP_K)
    return (logits, logits, logits, top_p, top_i)
